# Initial kernel scaffold; baseline (speedup 1.0000x reference)
import jax
import jax.numpy as jnp
from jax.experimental import pallas as pl


def kernel(observes, transitions):
    raise NotImplementedError("write your pallas kernel here")


# speedup vs baseline: 5.6343x; 5.6343x over previous
import functools

import jax
import jax.numpy as jnp
from jax.experimental import pallas as pl
from jax.experimental.pallas import tpu as pltpu


def _viterbi_kernel(obs_ref, tt_ref, out_ref, bp_ref, *, N, C, L):
    tt = tt_ref[...]

    def fwd_body(t, fvT):
        fvT3 = fvT[:, :, None]
        vit = fvT3[0] + tt[0][None, :]
        bp = jnp.zeros((N, C), dtype=jnp.int32)
        for p in range(1, C):
            s = fvT3[p] + tt[p][None, :]
            pred = s > vit
            vit = jnp.where(pred, s, vit)
            bp = jnp.where(pred, p, bp)
        bp_ref[t] = bp
        fv_new = vit + obs_ref[t]
        return fv_new.T

    fvT = jax.lax.fori_loop(0, L, fwd_body, jnp.zeros((C, N), jnp.float32))

    fvT3 = fvT[:, :, None]
    best = fvT3[0]
    end = jnp.zeros((N, 1), dtype=jnp.int32)
    for c in range(1, C):
        v = fvT3[c]
        pred = v > best
        best = jnp.where(pred, v, best)
        end = jnp.where(pred, c, end)

    lane = jax.lax.broadcasted_iota(jnp.int32, (N, C), 1)

    def back_body(i, bt):
        t = L - 1 - i
        bp_t = bp_ref[t]
        sel = jnp.where(lane == bt, bp_t, 0)
        new = jnp.max(sel, axis=1, keepdims=True)
        out_ref[t] = new[:, 0]
        return new

    jax.lax.fori_loop(0, L, back_body, end)


@jax.jit
def kernel(observes, transitions):
    N, C, L = observes.shape
    obs_t = jnp.transpose(observes, (2, 0, 1))
    tt = transitions.T
    path_t = pl.pallas_call(
        functools.partial(_viterbi_kernel, N=N, C=C, L=L),
        out_shape=jax.ShapeDtypeStruct((L, N), jnp.int32),
        in_specs=[
            pl.BlockSpec(memory_space=pltpu.VMEM),
            pl.BlockSpec(memory_space=pltpu.VMEM),
        ],
        out_specs=pl.BlockSpec(memory_space=pltpu.VMEM),
        scratch_shapes=[pltpu.VMEM((L, N, C), jnp.int32)],
    )(obs_t, tt)
    return path_t.T
